# Initial kernel scaffold; baseline (speedup 1.0000x reference)
import jax
import jax.numpy as jnp
from jax.experimental import pallas as pl


def kernel(x, pos, batch, params):
    raise NotImplementedError("write your pallas kernel here")


# speedup vs baseline: 4.4639x; 4.4639x over previous
import math

import jax
import jax.numpy as jnp
from jax.experimental import pallas as pl

_B = 4
_NEG_INF = float("-inf")


def _bn(z, gamma, beta):
    return z / math.sqrt(1.0 + 1e-5) * gamma + beta


def _body(x_ref, pos_ref, posT_ref, *rest):
    prefs = rest[:-1]
    out_ref = rest[-1]

    it = iter(prefs)

    def nxt_param():
        return next(it)[...]

    nets = {}
    for name in ("sa1", "sa2", "sa3", "fp3", "fp2", "fp1"):
        w1t, w1b, b1, g1, be1 = (nxt_param() for _ in range(5))
        w2, b2, g2, be2 = (nxt_param() for _ in range(4))
        nets[name] = (w1t, w1b, b1, g1, be1, w2, b2, g2, be2)
    lw1, lb1 = nxt_param(), nxt_param()
    lw2, lb2 = nxt_param(), nxt_param()
    lw3, lb3 = nxt_param(), nxt_param()

    xc = x_ref[0]
    pc = pos_ref[0]
    pcT = posT_ref[0]

    f32 = jnp.float32

    def fps(src, srcT, m, n):
        rows_m = jax.lax.broadcasted_iota(jnp.int32, (m, 1), 0)
        cols_m = jax.lax.broadcasted_iota(jnp.int32, (3, m), 1)
        iota_n = jax.lax.broadcasted_iota(jnp.int32, (n, 1), 0)
        iota_nT = jax.lax.broadcasted_iota(jnp.int32, (1, n), 1)
        d = jnp.sum((src - src[0:1, :]) ** 2, axis=1, keepdims=True)
        q = jnp.where(rows_m == 0, src[0:1, :], 0.0)
        qT = jnp.where(cols_m == 0, srcT[:, 0:1], 0.0)

        def body(i, st):
            d, q, qT = st
            mval = jnp.max(d)
            sel = jnp.min(jnp.where(d == mval, iota_n, n))
            p = jnp.sum(jnp.where(iota_n == sel, src, 0.0), axis=0,
                        keepdims=True)
            pT = jnp.sum(jnp.where(iota_nT == sel, srcT, 0.0), axis=1,
                         keepdims=True)
            q = jnp.where(rows_m == i, p, q)
            qT = jnp.where(cols_m == i, pT, qT)
            d = jnp.minimum(d, jnp.sum((src - p) ** 2, axis=1,
                                       keepdims=True))
            return d, q, qT

        _, q, qT = jax.lax.fori_loop(1, m, body, (d, q, qT))
        return q, qT

    def pair_d2(dst, srcT, nd, ns):
        d2 = None
        for c in range(3):
            t = dst[:, c:c + 1] - srcT[c:c + 1, :]
            d2 = t * t if d2 is None else d2 + t * t
        return d2

    def point_conv(q, srcT, a_src, c_q, w2, b2, g1, be1, g2, be2, r2,
                   nq, ns, cout):
        d2 = pair_d2(q, srcT, nq, ns)
        score = jnp.where(d2 <= r2, -d2, _NEG_INF)
        iota2 = jax.lax.broadcasted_iota(jnp.int32, (nq, ns), 1)
        acc0 = jnp.full((nq, cout), _NEG_INF, f32)

        def body(k, st):
            score, acc = st
            rowmax = jnp.max(score, axis=1, keepdims=True)
            valid = rowmax > _NEG_INF
            amin = jnp.min(jnp.where(score == rowmax, iota2, ns), axis=1,
                           keepdims=True)
            oh = iota2 == amin
            g = jnp.dot(oh.astype(f32), a_src,
                        preferred_element_type=f32)
            h = jnp.maximum(g - c_q, 0.0)
            h = _bn(h, g1, be1)
            h = jnp.maximum(
                jnp.dot(h, w2, preferred_element_type=f32) + b2, 0.0)
            h = _bn(h, g2, be2)
            acc = jnp.where(valid, jnp.maximum(acc, h), acc)
            score = jnp.where(oh, _NEG_INF, score)
            return score, acc

        _, acc = jax.lax.fori_loop(0, 64, body, (score, acc0))
        return acc

    def knn3(dst, srcT_pos, src_feat, nd, ns):
        d2 = pair_d2(dst, srcT_pos, nd, ns)
        iota2 = jax.lax.broadcasted_iota(jnp.int32, (nd, ns), 1)
        num = jnp.zeros((nd, src_feat.shape[1]), f32)
        den = jnp.zeros((nd, 1), f32)
        for _ in range(3):
            rowmin = jnp.min(d2, axis=1, keepdims=True)
            amin = jnp.min(jnp.where(d2 == rowmin, iota2, ns), axis=1,
                           keepdims=True)
            oh = iota2 == amin
            w = 1.0 / jnp.maximum(rowmin, 1e-16)
            num = num + w * jnp.dot(oh.astype(f32), src_feat,
                                    preferred_element_type=f32)
            den = den + w
            d2 = jnp.where(oh, float("inf"), d2)
        return num / den

    n = xc.shape[0]
    m1 = math.ceil(0.2 * n)
    m2 = math.ceil(0.25 * m1)

    q1, q1T = fps(pc, pcT, m1, n)
    w1t, w1b, b1, g1, be1, w2, b2, g2, be2 = nets["sa1"]
    a_src = (jnp.dot(xc, w1t, preferred_element_type=f32)
             + jnp.dot(pc, w1b, preferred_element_type=f32) + b1)
    c_q = jnp.dot(q1, w1b, preferred_element_type=f32)
    x1 = point_conv(q1, pcT, a_src, c_q, w2, b2, g1, be1, g2, be2,
                    0.2 * 0.2, m1, n, 64)

    q2, q2T = fps(q1, q1T, m2, m1)
    w1t, w1b, b1, g1, be1, w2, b2, g2, be2 = nets["sa2"]
    a_src = (jnp.dot(x1, w1t, preferred_element_type=f32)
             + jnp.dot(q1, w1b, preferred_element_type=f32) + b1)
    c_q = jnp.dot(q2, w1b, preferred_element_type=f32)
    x2 = point_conv(q2, q1T, a_src, c_q, w2, b2, g1, be1, g2, be2,
                    0.4 * 0.4, m2, m1, 128)

    w1t, w1b, b1, g1, be1, w2, b2, g2, be2 = nets["sa3"]
    z = (jnp.dot(x2, w1t, preferred_element_type=f32)
         + jnp.dot(q2, w1b, preferred_element_type=f32) + b1)
    z = _bn(jnp.maximum(z, 0.0), g1, be1)
    z = jnp.dot(z, w2, preferred_element_type=f32) + b2
    z = _bn(jnp.maximum(z, 0.0), g2, be2)
    x3 = jnp.max(z, axis=0, keepdims=True)

    w1t, w1b, b1, g1, be1, w2, b2, g2, be2 = nets["fp3"]
    z = (jnp.dot(x3, w1t, preferred_element_type=f32)
         + jnp.dot(x2, w1b, preferred_element_type=f32) + b1)
    z = _bn(jnp.maximum(z, 0.0), g1, be1)
    z = jnp.dot(z, w2, preferred_element_type=f32) + b2
    y3 = _bn(jnp.maximum(z, 0.0), g2, be2)

    yi = knn3(q1, q2T, y3, m1, m2)
    w1t, w1b, b1, g1, be1, w2, b2, g2, be2 = nets["fp2"]
    z = (jnp.dot(yi, w1t, preferred_element_type=f32)
         + jnp.dot(x1, w1b, preferred_element_type=f32) + b1)
    z = _bn(jnp.maximum(z, 0.0), g1, be1)
    z = jnp.dot(z, w2, preferred_element_type=f32) + b2
    y2 = _bn(jnp.maximum(z, 0.0), g2, be2)

    yi = knn3(pc, q1T, y2, n, m1)
    w1t, w1b, b1, g1, be1, w2, b2, g2, be2 = nets["fp1"]
    z = (jnp.dot(yi, w1t, preferred_element_type=f32)
         + jnp.dot(xc, w1b, preferred_element_type=f32) + b1)
    z = _bn(jnp.maximum(z, 0.0), g1, be1)
    z = jnp.dot(z, w2, preferred_element_type=f32) + b2
    y1 = _bn(jnp.maximum(z, 0.0), g2, be2)

    h = jnp.maximum(jnp.dot(y1, lw1, preferred_element_type=f32) + lb1, 0.0)
    h = jnp.dot(h, lw2, preferred_element_type=f32) + lb2
    h = jnp.dot(h, lw3, preferred_element_type=f32) + lb3
    mx = jnp.max(h, axis=-1, keepdims=True)
    s = h - mx
    out = s - jnp.log(jnp.sum(jnp.exp(s), axis=-1, keepdims=True))
    out_ref[0] = out


def kernel(x, pos, batch, params):
    N = x.shape[0]
    n = N // _B
    ncls = params["lin3"]["W"].shape[1]
    xb = x.reshape(_B, n, 3)
    pb = pos.reshape(_B, n, 3)
    pbT = pb.transpose(0, 2, 1)

    splits = {"sa1": 3, "sa2": 64, "sa3": 128, "fp3": 256, "fp2": 128,
              "fp1": 64}
    flat = []
    for name in ("sa1", "sa2", "sa3", "fp3", "fp2", "fp1"):
        l0, l1 = params[name]
        k = splits[name]
        flat += [l0["W"][:k], l0["W"][k:], l0["b"].reshape(1, -1),
                 l0["gamma"].reshape(1, -1), l0["beta"].reshape(1, -1)]
        flat += [l1["W"], l1["b"].reshape(1, -1),
                 l1["gamma"].reshape(1, -1), l1["beta"].reshape(1, -1)]
    for name in ("lin1", "lin2", "lin3"):
        flat += [params[name]["W"], params[name]["b"].reshape(1, -1)]

    def bcast_spec(a):
        return pl.BlockSpec(a.shape, lambda b: (0,) * a.ndim)

    in_specs = [
        pl.BlockSpec((1, n, 3), lambda b: (b, 0, 0)),
        pl.BlockSpec((1, n, 3), lambda b: (b, 0, 0)),
        pl.BlockSpec((1, 3, n), lambda b: (b, 0, 0)),
    ] + [bcast_spec(a) for a in flat]

    out = pl.pallas_call(
        _body,
        grid=(_B,),
        in_specs=in_specs,
        out_specs=pl.BlockSpec((1, n, ncls), lambda b: (b, 0, 0)),
        out_shape=jax.ShapeDtypeStruct((_B, n, ncls), jnp.float32),
    )(xb, pb, pbT, *flat)
    return out.reshape(N, ncls)


# speedup vs baseline: 8.7217x; 1.9538x over previous
import math

import jax
import jax.numpy as jnp
from jax.experimental import pallas as pl
from jax.experimental.pallas import tpu as pltpu

_B = 4
_NEG_INF = float("-inf")


def _bn(z, gamma, beta):
    return z / math.sqrt(1.0 + 1e-5) * gamma + beta


def _body(x_ref, pos_ref, posT_ref, *rest):
    prefs = rest[:-1]
    out_ref = rest[-1]

    it = iter(prefs)

    def nxt_param():
        return next(it)[...]

    nets = {}
    for name in ("sa1", "sa2", "sa3", "fp3", "fp2", "fp1"):
        w1t, w1b, b1, g1, be1 = (nxt_param() for _ in range(5))
        w2, b2, g2, be2 = (nxt_param() for _ in range(4))
        nets[name] = (w1t, w1b, b1, g1, be1, w2, b2, g2, be2)
    lw1, lb1 = nxt_param(), nxt_param()
    lw2, lb2 = nxt_param(), nxt_param()
    lw3, lb3 = nxt_param(), nxt_param()

    xc = x_ref[0]
    pc = pos_ref[0]
    pcT = posT_ref[0]

    f32 = jnp.float32

    def fps(srcT, m, n):
        cols_m = jax.lax.broadcasted_iota(jnp.int32, (3, m), 1)
        iota_n = jax.lax.broadcasted_iota(jnp.int32, (1, n), 1)
        p0 = srcT[:, 0:1]
        d = jnp.sum((srcT - p0) ** 2, axis=0, keepdims=True)
        qT = jnp.where(cols_m == 0, p0, 0.0)

        def body(i, st):
            d, qT = st
            mval = jnp.max(d)
            sel = jnp.min(jnp.where(d == mval, iota_n, n))
            pT = jnp.sum(jnp.where(iota_n == sel, srcT, 0.0), axis=1,
                         keepdims=True)
            qT = jnp.where(cols_m == i, pT, qT)
            d = jnp.minimum(d, jnp.sum((srcT - pT) ** 2, axis=0,
                                       keepdims=True))
            return d, qT

        _, qT = jax.lax.fori_loop(1, m, body, (d, qT))
        return jnp.transpose(qT), qT

    def pair_d2(dst, srcT, nd, ns):
        d2 = None
        for c in range(3):
            t = dst[:, c:c + 1] - srcT[c:c + 1, :]
            d2 = t * t if d2 is None else d2 + t * t
        return d2

    def point_conv(q, srcT, a_src, c_q, w2, b2, g1, be1, g2, be2, r2,
                   nq, ns, cout):
        d2 = pair_d2(q, srcT, nq, ns)
        score = jnp.where(d2 <= r2, -d2, _NEG_INF)
        iota2 = jax.lax.broadcasted_iota(jnp.int32, (nq, ns), 1)
        acc0 = jnp.full((nq, cout), _NEG_INF, f32)

        def body(k, st):
            score, acc = st
            rowmax = jnp.max(score, axis=1, keepdims=True)
            valid = rowmax > _NEG_INF
            amin = jnp.min(jnp.where(score == rowmax, iota2, ns), axis=1,
                           keepdims=True)
            oh = iota2 == amin
            g = jnp.dot(oh.astype(f32), a_src,
                        preferred_element_type=f32)
            h = jnp.maximum(g - c_q, 0.0)
            h = _bn(h, g1, be1)
            h = jnp.maximum(
                jnp.dot(h, w2, preferred_element_type=f32) + b2, 0.0)
            h = _bn(h, g2, be2)
            acc = jnp.where(valid, jnp.maximum(acc, h), acc)
            score = jnp.where(oh, _NEG_INF, score)
            return score, acc

        _, acc = jax.lax.fori_loop(0, 64, body, (score, acc0))
        return acc

    def knn3(dst, srcT_pos, src_feat, nd, ns):
        d2 = pair_d2(dst, srcT_pos, nd, ns)
        iota2 = jax.lax.broadcasted_iota(jnp.int32, (nd, ns), 1)
        num = jnp.zeros((nd, src_feat.shape[1]), f32)
        den = jnp.zeros((nd, 1), f32)
        for _ in range(3):
            rowmin = jnp.min(d2, axis=1, keepdims=True)
            amin = jnp.min(jnp.where(d2 == rowmin, iota2, ns), axis=1,
                           keepdims=True)
            oh = iota2 == amin
            w = 1.0 / jnp.maximum(rowmin, 1e-16)
            num = num + w * jnp.dot(oh.astype(f32), src_feat,
                                    preferred_element_type=f32)
            den = den + w
            d2 = jnp.where(oh, float("inf"), d2)
        return num / den

    n = xc.shape[0]
    m1 = math.ceil(0.2 * n)
    m2 = math.ceil(0.25 * m1)

    q1, q1T = fps(pcT, m1, n)
    w1t, w1b, b1, g1, be1, w2, b2, g2, be2 = nets["sa1"]
    a_src = (jnp.dot(xc, w1t, preferred_element_type=f32)
             + jnp.dot(pc, w1b, preferred_element_type=f32) + b1)
    c_q = jnp.dot(q1, w1b, preferred_element_type=f32)
    x1 = point_conv(q1, pcT, a_src, c_q, w2, b2, g1, be1, g2, be2,
                    0.2 * 0.2, m1, n, 64)

    q2, q2T = fps(q1T, m2, m1)
    w1t, w1b, b1, g1, be1, w2, b2, g2, be2 = nets["sa2"]
    a_src = (jnp.dot(x1, w1t, preferred_element_type=f32)
             + jnp.dot(q1, w1b, preferred_element_type=f32) + b1)
    c_q = jnp.dot(q2, w1b, preferred_element_type=f32)
    x2 = point_conv(q2, q1T, a_src, c_q, w2, b2, g1, be1, g2, be2,
                    0.4 * 0.4, m2, m1, 128)

    w1t, w1b, b1, g1, be1, w2, b2, g2, be2 = nets["sa3"]
    z = (jnp.dot(x2, w1t, preferred_element_type=f32)
         + jnp.dot(q2, w1b, preferred_element_type=f32) + b1)
    z = _bn(jnp.maximum(z, 0.0), g1, be1)
    z = jnp.dot(z, w2, preferred_element_type=f32) + b2
    z = _bn(jnp.maximum(z, 0.0), g2, be2)
    x3 = jnp.max(z, axis=0, keepdims=True)

    w1t, w1b, b1, g1, be1, w2, b2, g2, be2 = nets["fp3"]
    z = (jnp.dot(x3, w1t, preferred_element_type=f32)
         + jnp.dot(x2, w1b, preferred_element_type=f32) + b1)
    z = _bn(jnp.maximum(z, 0.0), g1, be1)
    z = jnp.dot(z, w2, preferred_element_type=f32) + b2
    y3 = _bn(jnp.maximum(z, 0.0), g2, be2)

    yi = knn3(q1, q2T, y3, m1, m2)
    w1t, w1b, b1, g1, be1, w2, b2, g2, be2 = nets["fp2"]
    z = (jnp.dot(yi, w1t, preferred_element_type=f32)
         + jnp.dot(x1, w1b, preferred_element_type=f32) + b1)
    z = _bn(jnp.maximum(z, 0.0), g1, be1)
    z = jnp.dot(z, w2, preferred_element_type=f32) + b2
    y2 = _bn(jnp.maximum(z, 0.0), g2, be2)

    yi = knn3(pc, q1T, y2, n, m1)
    w1t, w1b, b1, g1, be1, w2, b2, g2, be2 = nets["fp1"]
    z = (jnp.dot(yi, w1t, preferred_element_type=f32)
         + jnp.dot(xc, w1b, preferred_element_type=f32) + b1)
    z = _bn(jnp.maximum(z, 0.0), g1, be1)
    z = jnp.dot(z, w2, preferred_element_type=f32) + b2
    y1 = _bn(jnp.maximum(z, 0.0), g2, be2)

    h = jnp.maximum(jnp.dot(y1, lw1, preferred_element_type=f32) + lb1, 0.0)
    h = jnp.dot(h, lw2, preferred_element_type=f32) + lb2
    h = jnp.dot(h, lw3, preferred_element_type=f32) + lb3
    mx = jnp.max(h, axis=-1, keepdims=True)
    s = h - mx
    out = s - jnp.log(jnp.sum(jnp.exp(s), axis=-1, keepdims=True))
    out_ref[0] = out


def kernel(x, pos, batch, params):
    N = x.shape[0]
    n = N // _B
    ncls = params["lin3"]["W"].shape[1]
    xb = x.reshape(_B, n, 3)
    pb = pos.reshape(_B, n, 3)
    pbT = pb.transpose(0, 2, 1)

    splits = {"sa1": 3, "sa2": 64, "sa3": 128, "fp3": 256, "fp2": 128,
              "fp1": 64}
    flat = []
    for name in ("sa1", "sa2", "sa3", "fp3", "fp2", "fp1"):
        l0, l1 = params[name]
        k = splits[name]
        flat += [l0["W"][:k], l0["W"][k:], l0["b"].reshape(1, -1),
                 l0["gamma"].reshape(1, -1), l0["beta"].reshape(1, -1)]
        flat += [l1["W"], l1["b"].reshape(1, -1),
                 l1["gamma"].reshape(1, -1), l1["beta"].reshape(1, -1)]
    for name in ("lin1", "lin2", "lin3"):
        flat += [params[name]["W"], params[name]["b"].reshape(1, -1)]

    def bcast_spec(a):
        return pl.BlockSpec(a.shape, lambda b: (0,) * a.ndim)

    in_specs = [
        pl.BlockSpec((1, n, 3), lambda b: (b, 0, 0)),
        pl.BlockSpec((1, n, 3), lambda b: (b, 0, 0)),
        pl.BlockSpec((1, 3, n), lambda b: (b, 0, 0)),
    ] + [bcast_spec(a) for a in flat]

    out = pl.pallas_call(
        _body,
        grid=(_B,),
        in_specs=in_specs,
        out_specs=pl.BlockSpec((1, n, ncls), lambda b: (b, 0, 0)),
        out_shape=jax.ShapeDtypeStruct((_B, n, ncls), jnp.float32),
        compiler_params=pltpu.CompilerParams(
            dimension_semantics=("parallel",)),
    )(xb, pb, pbT, *flat)
    return out.reshape(N, ncls)


# speedup vs baseline: 9.6360x; 1.1048x over previous
import math

import jax
import jax.numpy as jnp
from jax.experimental import pallas as pl
from jax.experimental.pallas import tpu as pltpu

_B = 4
_NEG_INF = float("-inf")


def _bn(z, gamma, beta):
    return z / math.sqrt(1.0 + 1e-5) * gamma + beta


def _body(x_ref, pos_ref, posT_ref, *rest):
    prefs = rest[:-1]
    out_ref = rest[-1]

    it = iter(prefs)

    def nxt_param():
        return next(it)[...]

    nets = {}
    for name in ("sa1", "sa2", "sa3", "fp3", "fp2", "fp1"):
        w1t, w1b, b1, g1, be1 = (nxt_param() for _ in range(5))
        w2, b2, g2, be2 = (nxt_param() for _ in range(4))
        nets[name] = (w1t, w1b, b1, g1, be1, w2, b2, g2, be2)
    lw1, lb1 = nxt_param(), nxt_param()
    lw2, lb2 = nxt_param(), nxt_param()
    lw3, lb3 = nxt_param(), nxt_param()

    xc = x_ref[0]
    pc = pos_ref[0]
    pcT = posT_ref[0]

    f32 = jnp.float32

    def fps(srcT, m, n):
        cols_m = jax.lax.broadcasted_iota(jnp.int32, (3, m), 1)
        iota_n = jax.lax.broadcasted_iota(jnp.int32, (1, n), 1)
        p0 = srcT[:, 0:1]
        d = jnp.sum((srcT - p0) ** 2, axis=0, keepdims=True)
        qT = jnp.where(cols_m == 0, p0, 0.0)

        def body(i, st):
            d, qT = st
            mval = jnp.max(d)
            sel = jnp.min(jnp.where(d == mval, iota_n, n))
            pT = jnp.sum(jnp.where(iota_n == sel, srcT, 0.0), axis=1,
                         keepdims=True)
            qT = jnp.where(cols_m == i, pT, qT)
            d = jnp.minimum(d, jnp.sum((srcT - pT) ** 2, axis=0,
                                       keepdims=True))
            return d, qT

        _, qT = jax.lax.fori_loop(1, m, body, (d, qT), unroll=8)
        return jnp.transpose(qT), qT

    def pair_d2(dst, srcT, nd, ns):
        d2 = None
        for c in range(3):
            t = dst[:, c:c + 1] - srcT[c:c + 1, :]
            d2 = t * t if d2 is None else d2 + t * t
        return d2

    def point_conv(q, srcT, a_src, c_q, w2, b2, g1, be1, g2, be2, r2,
                   nq, ns, cout):
        d2 = pair_d2(q, srcT, nq, ns)
        score = jnp.where(d2 <= r2, -d2, _NEG_INF)
        iota2 = jax.lax.broadcasted_iota(jnp.int32, (nq, ns), 1)
        acc0 = jnp.full((nq, cout), _NEG_INF, f32)

        def body(k, st):
            score, acc = st
            rowmax = jnp.max(score, axis=1, keepdims=True)
            valid = rowmax > _NEG_INF
            amin = jnp.min(jnp.where(score == rowmax, iota2, ns), axis=1,
                           keepdims=True)
            oh = iota2 == amin
            g = jnp.dot(oh.astype(f32), a_src,
                        preferred_element_type=f32)
            h = jnp.maximum(g - c_q, 0.0)
            h = _bn(h, g1, be1)
            h = jnp.maximum(
                jnp.dot(h, w2, preferred_element_type=f32) + b2, 0.0)
            h = _bn(h, g2, be2)
            acc = jnp.where(valid, jnp.maximum(acc, h), acc)
            score = jnp.where(oh, _NEG_INF, score)
            return score, acc

        _, acc = jax.lax.fori_loop(0, 64, body, (score, acc0), unroll=4)
        return acc

    def knn3(dst, srcT_pos, src_feat, nd, ns):
        d2 = pair_d2(dst, srcT_pos, nd, ns)
        iota2 = jax.lax.broadcasted_iota(jnp.int32, (nd, ns), 1)
        num = jnp.zeros((nd, src_feat.shape[1]), f32)
        den = jnp.zeros((nd, 1), f32)
        for _ in range(3):
            rowmin = jnp.min(d2, axis=1, keepdims=True)
            amin = jnp.min(jnp.where(d2 == rowmin, iota2, ns), axis=1,
                           keepdims=True)
            oh = iota2 == amin
            w = 1.0 / jnp.maximum(rowmin, 1e-16)
            num = num + w * jnp.dot(oh.astype(f32), src_feat,
                                    preferred_element_type=f32)
            den = den + w
            d2 = jnp.where(oh, float("inf"), d2)
        return num / den

    n = xc.shape[0]
    m1 = math.ceil(0.2 * n)
    m2 = math.ceil(0.25 * m1)

    q1, q1T = fps(pcT, m1, n)
    w1t, w1b, b1, g1, be1, w2, b2, g2, be2 = nets["sa1"]
    a_src = (jnp.dot(xc, w1t, preferred_element_type=f32)
             + jnp.dot(pc, w1b, preferred_element_type=f32) + b1)
    c_q = jnp.dot(q1, w1b, preferred_element_type=f32)
    x1 = point_conv(q1, pcT, a_src, c_q, w2, b2, g1, be1, g2, be2,
                    0.2 * 0.2, m1, n, 64)

    q2, q2T = fps(q1T, m2, m1)
    w1t, w1b, b1, g1, be1, w2, b2, g2, be2 = nets["sa2"]
    a_src = (jnp.dot(x1, w1t, preferred_element_type=f32)
             + jnp.dot(q1, w1b, preferred_element_type=f32) + b1)
    c_q = jnp.dot(q2, w1b, preferred_element_type=f32)
    x2 = point_conv(q2, q1T, a_src, c_q, w2, b2, g1, be1, g2, be2,
                    0.4 * 0.4, m2, m1, 128)

    w1t, w1b, b1, g1, be1, w2, b2, g2, be2 = nets["sa3"]
    z = (jnp.dot(x2, w1t, preferred_element_type=f32)
         + jnp.dot(q2, w1b, preferred_element_type=f32) + b1)
    z = _bn(jnp.maximum(z, 0.0), g1, be1)
    z = jnp.dot(z, w2, preferred_element_type=f32) + b2
    z = _bn(jnp.maximum(z, 0.0), g2, be2)
    x3 = jnp.max(z, axis=0, keepdims=True)

    w1t, w1b, b1, g1, be1, w2, b2, g2, be2 = nets["fp3"]
    z = (jnp.dot(x3, w1t, preferred_element_type=f32)
         + jnp.dot(x2, w1b, preferred_element_type=f32) + b1)
    z = _bn(jnp.maximum(z, 0.0), g1, be1)
    z = jnp.dot(z, w2, preferred_element_type=f32) + b2
    y3 = _bn(jnp.maximum(z, 0.0), g2, be2)

    yi = knn3(q1, q2T, y3, m1, m2)
    w1t, w1b, b1, g1, be1, w2, b2, g2, be2 = nets["fp2"]
    z = (jnp.dot(yi, w1t, preferred_element_type=f32)
         + jnp.dot(x1, w1b, preferred_element_type=f32) + b1)
    z = _bn(jnp.maximum(z, 0.0), g1, be1)
    z = jnp.dot(z, w2, preferred_element_type=f32) + b2
    y2 = _bn(jnp.maximum(z, 0.0), g2, be2)

    yi = knn3(pc, q1T, y2, n, m1)
    w1t, w1b, b1, g1, be1, w2, b2, g2, be2 = nets["fp1"]
    z = (jnp.dot(yi, w1t, preferred_element_type=f32)
         + jnp.dot(xc, w1b, preferred_element_type=f32) + b1)
    z = _bn(jnp.maximum(z, 0.0), g1, be1)
    z = jnp.dot(z, w2, preferred_element_type=f32) + b2
    y1 = _bn(jnp.maximum(z, 0.0), g2, be2)

    h = jnp.maximum(jnp.dot(y1, lw1, preferred_element_type=f32) + lb1, 0.0)
    h = jnp.dot(h, lw2, preferred_element_type=f32) + lb2
    h = jnp.dot(h, lw3, preferred_element_type=f32) + lb3
    mx = jnp.max(h, axis=-1, keepdims=True)
    s = h - mx
    out = s - jnp.log(jnp.sum(jnp.exp(s), axis=-1, keepdims=True))
    out_ref[0] = out


def kernel(x, pos, batch, params):
    N = x.shape[0]
    n = N // _B
    ncls = params["lin3"]["W"].shape[1]
    xb = x.reshape(_B, n, 3)
    pb = pos.reshape(_B, n, 3)
    pbT = pb.transpose(0, 2, 1)

    splits = {"sa1": 3, "sa2": 64, "sa3": 128, "fp3": 256, "fp2": 128,
              "fp1": 64}
    flat = []
    for name in ("sa1", "sa2", "sa3", "fp3", "fp2", "fp1"):
        l0, l1 = params[name]
        k = splits[name]
        flat += [l0["W"][:k], l0["W"][k:], l0["b"].reshape(1, -1),
                 l0["gamma"].reshape(1, -1), l0["beta"].reshape(1, -1)]
        flat += [l1["W"], l1["b"].reshape(1, -1),
                 l1["gamma"].reshape(1, -1), l1["beta"].reshape(1, -1)]
    for name in ("lin1", "lin2", "lin3"):
        flat += [params[name]["W"], params[name]["b"].reshape(1, -1)]

    def bcast_spec(a):
        return pl.BlockSpec(a.shape, lambda b: (0,) * a.ndim)

    in_specs = [
        pl.BlockSpec((1, n, 3), lambda b: (b, 0, 0)),
        pl.BlockSpec((1, n, 3), lambda b: (b, 0, 0)),
        pl.BlockSpec((1, 3, n), lambda b: (b, 0, 0)),
    ] + [bcast_spec(a) for a in flat]

    out = pl.pallas_call(
        _body,
        grid=(_B,),
        in_specs=in_specs,
        out_specs=pl.BlockSpec((1, n, ncls), lambda b: (b, 0, 0)),
        out_shape=jax.ShapeDtypeStruct((_B, n, ncls), jnp.float32),
        compiler_params=pltpu.CompilerParams(
            dimension_semantics=("parallel",)),
    )(xb, pb, pbT, *flat)
    return out.reshape(N, ncls)


# speedup vs baseline: 9.8927x; 1.0266x over previous
import math

import jax
import jax.numpy as jnp
from jax.experimental import pallas as pl
from jax.experimental.pallas import tpu as pltpu

_B = 4
_NEG_INF = float("-inf")


def _bn(z, gamma, beta):
    return z / math.sqrt(1.0 + 1e-5) * gamma + beta


def _body(x_ref, pos_ref, posT_ref, *rest):
    prefs = rest[:-1]
    out_ref = rest[-1]

    it = iter(prefs)

    def nxt_param():
        return next(it)[...]

    nets = {}
    for name in ("sa1", "sa2", "sa3", "fp3", "fp2", "fp1"):
        w1t, w1b, b1, g1, be1 = (nxt_param() for _ in range(5))
        w2, b2, g2, be2 = (nxt_param() for _ in range(4))
        nets[name] = (w1t, w1b, b1, g1, be1, w2, b2, g2, be2)
    lw1, lb1 = nxt_param(), nxt_param()
    lw2, lb2 = nxt_param(), nxt_param()
    lw3, lb3 = nxt_param(), nxt_param()

    xc = x_ref[0]
    pc = pos_ref[0]
    pcT = posT_ref[0]

    f32 = jnp.float32

    def fps(srcT, m, n):
        cols_m = jax.lax.broadcasted_iota(jnp.int32, (3, m), 1)
        iota_n = jax.lax.broadcasted_iota(jnp.int32, (1, n), 1)
        p0 = srcT[:, 0:1]
        d = jnp.sum((srcT - p0) ** 2, axis=0, keepdims=True)
        qT = jnp.where(cols_m == 0, p0, 0.0)

        def body(i, st):
            d, qT = st
            mval = jnp.max(d)
            sel = jnp.min(jnp.where(d == mval, iota_n, n))
            pT = jnp.sum(jnp.where(iota_n == sel, srcT, 0.0), axis=1,
                         keepdims=True)
            qT = jnp.where(cols_m == i, pT, qT)
            d = jnp.minimum(d, jnp.sum((srcT - pT) ** 2, axis=0,
                                       keepdims=True))
            return d, qT

        _, qT = jax.lax.fori_loop(1, m, body, (d, qT), unroll=16)
        return jnp.transpose(qT), qT

    def pair_d2(dst, srcT, nd, ns):
        d2 = None
        for c in range(3):
            t = dst[:, c:c + 1] - srcT[c:c + 1, :]
            d2 = t * t if d2 is None else d2 + t * t
        return d2

    def point_conv(q, srcT, a_src, c_q, w2, b2, g1, be1, g2, be2, r2,
                   nq, ns, cout):
        d2 = pair_d2(q, srcT, nq, ns)
        score = jnp.where(d2 <= r2, -d2, _NEG_INF)
        iota2 = jax.lax.broadcasted_iota(jnp.int32, (nq, ns), 1)
        acc0 = jnp.full((nq, cout), _NEG_INF, f32)

        def body(k, st):
            score, acc = st
            rowmax = jnp.max(score, axis=1, keepdims=True)
            valid = rowmax > _NEG_INF
            amin = jnp.min(jnp.where(score == rowmax, iota2, ns), axis=1,
                           keepdims=True)
            oh = iota2 == amin
            g = jnp.dot(oh.astype(f32), a_src,
                        preferred_element_type=f32)
            h = jnp.maximum(g - c_q, 0.0)
            h = _bn(h, g1, be1)
            h = jnp.maximum(
                jnp.dot(h, w2, preferred_element_type=f32) + b2, 0.0)
            h = _bn(h, g2, be2)
            acc = jnp.where(valid, jnp.maximum(acc, h), acc)
            score = jnp.where(oh, _NEG_INF, score)
            return score, acc

        _, acc = jax.lax.fori_loop(0, 64, body, (score, acc0), unroll=8)
        return acc

    def knn3(dst, srcT_pos, src_feat, nd, ns):
        d2 = pair_d2(dst, srcT_pos, nd, ns)
        iota2 = jax.lax.broadcasted_iota(jnp.int32, (nd, ns), 1)
        num = jnp.zeros((nd, src_feat.shape[1]), f32)
        den = jnp.zeros((nd, 1), f32)
        for _ in range(3):
            rowmin = jnp.min(d2, axis=1, keepdims=True)
            amin = jnp.min(jnp.where(d2 == rowmin, iota2, ns), axis=1,
                           keepdims=True)
            oh = iota2 == amin
            w = 1.0 / jnp.maximum(rowmin, 1e-16)
            num = num + w * jnp.dot(oh.astype(f32), src_feat,
                                    preferred_element_type=f32)
            den = den + w
            d2 = jnp.where(oh, float("inf"), d2)
        return num / den

    n = xc.shape[0]
    m1 = math.ceil(0.2 * n)
    m2 = math.ceil(0.25 * m1)

    q1, q1T = fps(pcT, m1, n)
    w1t, w1b, b1, g1, be1, w2, b2, g2, be2 = nets["sa1"]
    a_src = (jnp.dot(xc, w1t, preferred_element_type=f32)
             + jnp.dot(pc, w1b, preferred_element_type=f32) + b1)
    c_q = jnp.dot(q1, w1b, preferred_element_type=f32)
    x1 = point_conv(q1, pcT, a_src, c_q, w2, b2, g1, be1, g2, be2,
                    0.2 * 0.2, m1, n, 64)

    q2, q2T = fps(q1T, m2, m1)
    w1t, w1b, b1, g1, be1, w2, b2, g2, be2 = nets["sa2"]
    a_src = (jnp.dot(x1, w1t, preferred_element_type=f32)
             + jnp.dot(q1, w1b, preferred_element_type=f32) + b1)
    c_q = jnp.dot(q2, w1b, preferred_element_type=f32)
    x2 = point_conv(q2, q1T, a_src, c_q, w2, b2, g1, be1, g2, be2,
                    0.4 * 0.4, m2, m1, 128)

    w1t, w1b, b1, g1, be1, w2, b2, g2, be2 = nets["sa3"]
    z = (jnp.dot(x2, w1t, preferred_element_type=f32)
         + jnp.dot(q2, w1b, preferred_element_type=f32) + b1)
    z = _bn(jnp.maximum(z, 0.0), g1, be1)
    z = jnp.dot(z, w2, preferred_element_type=f32) + b2
    z = _bn(jnp.maximum(z, 0.0), g2, be2)
    x3 = jnp.max(z, axis=0, keepdims=True)

    w1t, w1b, b1, g1, be1, w2, b2, g2, be2 = nets["fp3"]
    z = (jnp.dot(x3, w1t, preferred_element_type=f32)
         + jnp.dot(x2, w1b, preferred_element_type=f32) + b1)
    z = _bn(jnp.maximum(z, 0.0), g1, be1)
    z = jnp.dot(z, w2, preferred_element_type=f32) + b2
    y3 = _bn(jnp.maximum(z, 0.0), g2, be2)

    yi = knn3(q1, q2T, y3, m1, m2)
    w1t, w1b, b1, g1, be1, w2, b2, g2, be2 = nets["fp2"]
    z = (jnp.dot(yi, w1t, preferred_element_type=f32)
         + jnp.dot(x1, w1b, preferred_element_type=f32) + b1)
    z = _bn(jnp.maximum(z, 0.0), g1, be1)
    z = jnp.dot(z, w2, preferred_element_type=f32) + b2
    y2 = _bn(jnp.maximum(z, 0.0), g2, be2)

    yi = knn3(pc, q1T, y2, n, m1)
    w1t, w1b, b1, g1, be1, w2, b2, g2, be2 = nets["fp1"]
    z = (jnp.dot(yi, w1t, preferred_element_type=f32)
         + jnp.dot(xc, w1b, preferred_element_type=f32) + b1)
    z = _bn(jnp.maximum(z, 0.0), g1, be1)
    z = jnp.dot(z, w2, preferred_element_type=f32) + b2
    y1 = _bn(jnp.maximum(z, 0.0), g2, be2)

    h = jnp.maximum(jnp.dot(y1, lw1, preferred_element_type=f32) + lb1, 0.0)
    h = jnp.dot(h, lw2, preferred_element_type=f32) + lb2
    h = jnp.dot(h, lw3, preferred_element_type=f32) + lb3
    mx = jnp.max(h, axis=-1, keepdims=True)
    s = h - mx
    out = s - jnp.log(jnp.sum(jnp.exp(s), axis=-1, keepdims=True))
    out_ref[0] = out


def kernel(x, pos, batch, params):
    N = x.shape[0]
    n = N // _B
    ncls = params["lin3"]["W"].shape[1]
    xb = x.reshape(_B, n, 3)
    pb = pos.reshape(_B, n, 3)
    pbT = pb.transpose(0, 2, 1)

    splits = {"sa1": 3, "sa2": 64, "sa3": 128, "fp3": 256, "fp2": 128,
              "fp1": 64}
    flat = []
    for name in ("sa1", "sa2", "sa3", "fp3", "fp2", "fp1"):
        l0, l1 = params[name]
        k = splits[name]
        flat += [l0["W"][:k], l0["W"][k:], l0["b"].reshape(1, -1),
                 l0["gamma"].reshape(1, -1), l0["beta"].reshape(1, -1)]
        flat += [l1["W"], l1["b"].reshape(1, -1),
                 l1["gamma"].reshape(1, -1), l1["beta"].reshape(1, -1)]
    for name in ("lin1", "lin2", "lin3"):
        flat += [params[name]["W"], params[name]["b"].reshape(1, -1)]

    def bcast_spec(a):
        return pl.BlockSpec(a.shape, lambda b: (0,) * a.ndim)

    in_specs = [
        pl.BlockSpec((1, n, 3), lambda b: (b, 0, 0)),
        pl.BlockSpec((1, n, 3), lambda b: (b, 0, 0)),
        pl.BlockSpec((1, 3, n), lambda b: (b, 0, 0)),
    ] + [bcast_spec(a) for a in flat]

    out = pl.pallas_call(
        _body,
        grid=(_B,),
        in_specs=in_specs,
        out_specs=pl.BlockSpec((1, n, ncls), lambda b: (b, 0, 0)),
        out_shape=jax.ShapeDtypeStruct((_B, n, ncls), jnp.float32),
        compiler_params=pltpu.CompilerParams(
            dimension_semantics=("parallel",)),
    )(xb, pb, pbT, *flat)
    return out.reshape(N, ncls)
